# baseline (speedup 1.0000x reference)
import math

import jax
import jax.numpy as jnp
from jax.experimental import pallas as pl


_BLK = 2048


def _body(noise_ref, means_ref, stds_ref, out_ref):
    a = noise_ref[...] * stds_ref[...] + means_ref[...]
    col = jax.lax.broadcasted_iota(jnp.int32, a.shape, 1)
    lengths = jnp.maximum(jnp.abs(a), 0.1)
    angles = jnp.clip(a, 0.1, math.pi - 0.1)
    pos = a - jnp.floor(a)
    sq = jnp.where(col >= 9, a * a, 0.0)
    norm = jnp.sqrt(jnp.sum(sq, axis=1, keepdims=True)) + 1e-8
    new_norm = jnp.clip(norm, 0.01, 2.0 * math.pi)
    rot = a * (new_norm / norm)
    out_ref[...] = jnp.where(
        col < 3,
        lengths,
        jnp.where(col < 6, angles, jnp.where(col < 9, pos, rot)),
    )


def kernel(num_samples, noise, sg_ind, means, stds):
    del sg_ind
    n, d = noise.shape
    means2 = means.reshape(1, d)
    stds2 = stds.reshape(1, d)
    grid = (n // _BLK,)
    return pl.pallas_call(
        _body,
        grid=grid,
        in_specs=[
            pl.BlockSpec((_BLK, d), lambda i: (i, 0)),
            pl.BlockSpec((1, d), lambda i: (0, 0)),
            pl.BlockSpec((1, d), lambda i: (0, 0)),
        ],
        out_specs=pl.BlockSpec((_BLK, d), lambda i: (i, 0)),
        out_shape=jax.ShapeDtypeStruct((n, d), noise.dtype),
    )(noise, means2, stds2)
